# baseline (speedup 1.0000x reference)
import functools

import jax
import jax.numpy as jnp
from jax import lax
from jax.experimental import pallas as pl
from jax.experimental.pallas import tpu as pltpu
from jax.experimental.pallas import tpu_sc as plsc

_NC = 2
_NS = 16
_NW = _NC * _NS
_CHUNK = 80
_G = 8
_CW = 16


def _round_up(a: int, b: int) -> int:
    return (a + b - 1) // b * b


@functools.lru_cache(maxsize=None)
def _make_sc_agg(n_pad: int, w: int, e_pad: int):
    epw = e_pad // _NW
    nchunk = epw // _CHUNK
    ngroups = nchunk // _G
    assert ngroups % 2 == 0
    rows_ps = n_pad // _NS
    assert rows_ps % 16 == 0

    mesh = plsc.VectorSubcoreMesh(core_axis_name="c", subcore_axis_name="s")
    scratch = [
        pltpu.VMEM((_G, 2, _CHUNK), jnp.int32),
        pltpu.VMEM((_G, 2, _CHUNK), jnp.int32),
        pltpu.VMEM((_CHUNK, w), jnp.float32),
        pltpu.VMEM((_CHUNK, w), jnp.float32),
        pltpu.VMEM((16, w), jnp.float32),
        pltpu.VMEM_SHARED((n_pad, w), jnp.float32),
        pltpu.SemaphoreType.DMA,
        pltpu.SemaphoreType.DMA,
        pltpu.SemaphoreType.DMA,
        pltpu.SemaphoreType.DMA,
        pltpu.SemaphoreType.DMA,
        pltpu.SemaphoreType.DMA,
    ]

    def body(e2_h, h_h, acc_o, iga, igb, r0, r1, zbuf, acc,
             sia, sib, sg0, sg1, ss0, ss1):
        c = lax.axis_index("c")
        s = lax.axis_index("s")
        wid = s * _NC + c

        z16 = jnp.zeros((16,), jnp.float32)
        for i in range(16):
            for j in range(w // 16):
                zbuf[i, pl.ds(j * 16, 16)] = z16

        base_row = s * rows_ps

        def zero_body(t, carry):
            pltpu.sync_copy(zbuf, acc.at[pl.ds(base_row + t * 16, 16)])
            return carry

        lax.fori_loop(0, rows_ps // 16, zero_body, 0)
        pltpu.sync_copy(e2_h.at[wid, pl.ds(0, _G)], iga)
        plsc.subcore_barrier()

        rbuf = (r0, r1)
        gsem = (sg0, sg1)
        ssem = (ss0, ss1)

        def process_group(ig):
            ds = [pltpu.async_copy(h_h.at[ig.at[k, 0]], r0, sg0)
                  for k in range(_G)]
            for d in ds:
                d.wait()

        def outer_body(t, carry):
            ga = 2 * t
            dpb = pltpu.async_copy(e2_h.at[wid, pl.ds((ga + 1) * _G, _G)],
                                   igb, sib)
            process_group(iga)
            dpb.wait()
            dpa = pltpu.async_copy(
                e2_h.at[wid, pl.ds(lax.rem((ga + 2), ngroups) * _G, _G)],
                iga, sia)
            process_group(igb)
            dpa.wait()
            return carry

        lax.fori_loop(0, ngroups // 2, outer_body, 0)
        plsc.subcore_barrier()

        pltpu.sync_copy(acc.at[pl.ds(base_row, rows_ps)],
                        acc_o.at[c, pl.ds(base_row, rows_ps)])

    return pl.kernel(
        body,
        out_type=jax.ShapeDtypeStruct((_NC, n_pad, w), jnp.float32),
        mesh=mesh, scratch_types=scratch,
        compiler_params=pltpu.CompilerParams(use_tc_tiling_on_sc=False))


def _tc_layer1(P, x, Wl, bl, Wr, block_rows):
    n, d = x.shape
    w = P.shape[2]

    def body(p_ref, x_ref, wl_ref, bl_ref, wr_ref, o_ref):
        agg = p_ref[0, :, :d] + p_ref[1, :, :d]
        cnt = jnp.mean(p_ref[0, :, d:] + p_ref[1, :, d:], axis=1, keepdims=True)
        mean = agg / jnp.maximum(cnt, 1.0)
        h = jnp.dot(mean, wl_ref[...], preferred_element_type=jnp.float32)
        h = h + bl_ref[...]
        h = h + jnp.dot(x_ref[...], wr_ref[...], preferred_element_type=jnp.float32)
        o_ref[...] = jnp.maximum(h, 0.0)

    return pl.pallas_call(
        body,
        grid=(n // block_rows,),
        in_specs=[
            pl.BlockSpec((_NC, block_rows, w), lambda i: (0, i, 0)),
            pl.BlockSpec((block_rows, d), lambda i: (i, 0)),
            pl.BlockSpec((d, d), lambda i: (0, 0)),
            pl.BlockSpec((1, d), lambda i: (0, 0)),
            pl.BlockSpec((d, d), lambda i: (0, 0)),
        ],
        out_specs=pl.BlockSpec((block_rows, d), lambda i: (i, 0)),
        out_shape=jax.ShapeDtypeStruct((n, d), jnp.float32),
    )(P, x, Wl, bl.reshape(1, d), Wr)


def _tc_layer2(P, Pw, h1, x, Wl, bl, Wr, gamma, beta, block_rows):
    n, d = x.shape
    w = Pw.shape[2]

    def body(p_ref, pw_ref, h_ref, x_ref, wl_ref, bl_ref, wr_ref, g_ref, b_ref,
             o_ref):
        agg = p_ref[0] + p_ref[1]
        cnt = jnp.mean(pw_ref[0, :, d:] + pw_ref[1, :, d:], axis=1,
                       keepdims=True)
        mean = agg / jnp.maximum(cnt, 1.0)
        h = jnp.dot(mean, wl_ref[...], preferred_element_type=jnp.float32)
        h = h + bl_ref[...]
        h = h + jnp.dot(h_ref[...], wr_ref[...], preferred_element_type=jnp.float32)
        h = h + x_ref[...]
        mu = jnp.mean(h, axis=1, keepdims=True)
        hc = h - mu
        var = jnp.mean(hc * hc, axis=1, keepdims=True)
        o_ref[...] = hc * lax.rsqrt(var + 1e-5) * g_ref[...] + b_ref[...]

    return pl.pallas_call(
        body,
        grid=(n // block_rows,),
        in_specs=[
            pl.BlockSpec((_NC, block_rows, d), lambda i: (0, i, 0)),
            pl.BlockSpec((_NC, block_rows, w), lambda i: (0, i, 0)),
            pl.BlockSpec((block_rows, d), lambda i: (i, 0)),
            pl.BlockSpec((block_rows, d), lambda i: (i, 0)),
            pl.BlockSpec((d, d), lambda i: (0, 0)),
            pl.BlockSpec((1, d), lambda i: (0, 0)),
            pl.BlockSpec((d, d), lambda i: (0, 0)),
            pl.BlockSpec((1, d), lambda i: (0, 0)),
            pl.BlockSpec((1, d), lambda i: (0, 0)),
        ],
        out_specs=pl.BlockSpec((block_rows, d), lambda i: (i, 0)),
        out_shape=jax.ShapeDtypeStruct((n, d), jnp.float32),
    )(P, Pw, h1, x, Wl, bl.reshape(1, d), Wr, gamma.reshape(1, d),
      beta.reshape(1, d))


def kernel(x, edge_index, W1l, b1l, W1r, W2l, b2l, W2r, gamma, beta):
    n, d = x.shape
    e = edge_index.shape[1]

    e_pad = _round_up(e, _NW * _CHUNK * _G * 2)
    n_pad = _round_up(n + 1, 16 * _NS)

    src = edge_index[0]
    dst = edge_index[1]
    if e_pad != e:
        pad = e_pad - e
        src = jnp.concatenate([src, jnp.zeros((pad,), jnp.int32)])
        dst = jnp.concatenate([dst, jnp.full((pad,), n, jnp.int32)])
    nchunk = e_pad // (_NW * _CHUNK)
    e2 = jnp.stack([src.reshape(_NW, nchunk, _CHUNK),
                    dst.reshape(_NW, nchunk, _CHUNK)], axis=2)

    xa = jnp.concatenate([x, jnp.ones((n, _CW), jnp.float32)], axis=1)

    P1w = _make_sc_agg(n_pad, d + _CW, e_pad)(e2, xa)
    h1 = _tc_layer1(P1w, x, W1l, b1l, W1r, 400)
    P2 = _make_sc_agg(n_pad, d, e_pad)(e2, h1)
    return _tc_layer2(P2, P1w, h1, x, W2l, b2l, W2r, gamma, beta, 400)


# speedup vs baseline: 1.9579x; 1.9579x over previous
import functools

import jax
import jax.numpy as jnp
from jax import lax
from jax.experimental import pallas as pl
from jax.experimental.pallas import tpu as pltpu
from jax.experimental.pallas import tpu_sc as plsc

_NC = 2
_NS = 16
_NW = _NC * _NS
_CHUNK = 128
_G = 8
_CW = 16


def _round_up(a: int, b: int) -> int:
    return (a + b - 1) // b * b


@functools.lru_cache(maxsize=None)
def _make_sc_agg(n: int, n_pad: int, w: int, e_pad: int):
    epw = e_pad // _NW
    nchunk = epw // _CHUNK
    ngroups = nchunk // _G
    assert ngroups % 2 == 0
    rows_ps = n_pad // _NS
    assert rows_ps % 16 == 0
    assert n % _NS == 0
    tbl_ps = n // _NS

    mesh = plsc.VectorSubcoreMesh(core_axis_name="c", subcore_axis_name="s")
    scratch = [
        pltpu.VMEM((_G, 2, _CHUNK), jnp.int32),
        pltpu.VMEM((_G, 2, _CHUNK), jnp.int32),
        pltpu.VMEM((_CHUNK, w), jnp.float32),
        pltpu.VMEM((_CHUNK, w), jnp.float32),
        pltpu.VMEM((16, w), jnp.float32),
        pltpu.VMEM_SHARED((n, w), jnp.float32),
        pltpu.VMEM_SHARED((n_pad, w), jnp.float32),
        pltpu.SemaphoreType.DMA,
        pltpu.SemaphoreType.DMA,
        pltpu.SemaphoreType.DMA,
        pltpu.SemaphoreType.DMA,
        pltpu.SemaphoreType.DMA,
        pltpu.SemaphoreType.DMA,
    ]

    def body(e2_h, h_h, acc_o, iga, igb, r0, r1, zbuf, tbl, acc,
             sia, sib, sg0, sg1, ss0, ss1):
        c = lax.axis_index("c")
        s = lax.axis_index("s")
        wid = s * _NC + c

        pltpu.sync_copy(h_h.at[pl.ds(s * tbl_ps, tbl_ps)],
                        tbl.at[pl.ds(s * tbl_ps, tbl_ps)])

        z16 = jnp.zeros((16,), jnp.float32)
        for i in range(16):
            for j in range(w // 16):
                zbuf[i, pl.ds(j * 16, 16)] = z16

        base_row = s * rows_ps

        def zero_body(t, carry):
            pltpu.sync_copy(zbuf, acc.at[pl.ds(base_row + t * 16, 16)])
            return carry

        lax.fori_loop(0, rows_ps // 16, zero_body, 0)
        pltpu.sync_copy(e2_h.at[wid, pl.ds(0, _G)], iga)
        plsc.subcore_barrier()

        def process_group(ig):
            dg = [None, None]
            dg[0] = pltpu.async_copy(tbl.at[ig.at[0, 0]], r0, sg0)
            for p in range(_G // 2):
                j0 = 2 * p
                dg[0].wait()
                ds0 = pltpu.async_copy(r0, acc.at[ig.at[j0, 1]], ss0, add=True)
                dg[1] = pltpu.async_copy(tbl.at[ig.at[j0 + 1, 0]], r1, sg1)
                ds0.wait()
                if j0 + 2 < _G:
                    dg[0] = pltpu.async_copy(tbl.at[ig.at[j0 + 2, 0]], r0, sg0)
                dg[1].wait()
                ds1 = pltpu.async_copy(r1, acc.at[ig.at[j0 + 1, 1]], ss1,
                                       add=True)
                ds1.wait()

        def outer_body(t, carry):
            ga = 2 * t
            dpb = pltpu.async_copy(e2_h.at[wid, pl.ds((ga + 1) * _G, _G)],
                                   igb, sib)
            process_group(iga)
            dpb.wait()
            dpa = pltpu.async_copy(
                e2_h.at[wid, pl.ds(lax.rem((ga + 2), ngroups) * _G, _G)],
                iga, sia)
            process_group(igb)
            dpa.wait()
            return carry

        lax.fori_loop(0, ngroups // 2, outer_body, 0)
        plsc.subcore_barrier()

        pltpu.sync_copy(acc.at[pl.ds(base_row, rows_ps)],
                        acc_o.at[c, pl.ds(base_row, rows_ps)])

    return pl.kernel(
        body,
        out_type=jax.ShapeDtypeStruct((_NC, n_pad, w), jnp.float32),
        mesh=mesh, scratch_types=scratch,
        compiler_params=pltpu.CompilerParams(use_tc_tiling_on_sc=False))


def _tc_layer1(Pa, Pb, x, Wl, bl, Wr, block_rows):
    n, d = x.shape
    wa = Pa.shape[2]
    wb = Pb.shape[2]
    db = d - wa

    def body(pa_ref, pb_ref, x_ref, wl_ref, bl_ref, wr_ref, o_ref):
        agg = jnp.concatenate(
            [pa_ref[0] + pa_ref[1],
             pb_ref[0, :, :db] + pb_ref[1, :, :db]], axis=1)
        cnt = jnp.mean(pb_ref[0, :, db:] + pb_ref[1, :, db:], axis=1,
                       keepdims=True)
        mean = agg / jnp.maximum(cnt, 1.0)
        h = jnp.dot(mean, wl_ref[...], preferred_element_type=jnp.float32)
        h = h + bl_ref[...]
        h = h + jnp.dot(x_ref[...], wr_ref[...], preferred_element_type=jnp.float32)
        o_ref[...] = jnp.maximum(h, 0.0)

    return pl.pallas_call(
        body,
        grid=(n // block_rows,),
        in_specs=[
            pl.BlockSpec((_NC, block_rows, wa), lambda i: (0, i, 0)),
            pl.BlockSpec((_NC, block_rows, wb), lambda i: (0, i, 0)),
            pl.BlockSpec((block_rows, d), lambda i: (i, 0)),
            pl.BlockSpec((d, d), lambda i: (0, 0)),
            pl.BlockSpec((1, d), lambda i: (0, 0)),
            pl.BlockSpec((d, d), lambda i: (0, 0)),
        ],
        out_specs=pl.BlockSpec((block_rows, d), lambda i: (i, 0)),
        out_shape=jax.ShapeDtypeStruct((n, d), jnp.float32),
    )(Pa, Pb, x, Wl, bl.reshape(1, d), Wr)


def _tc_layer2(Pa, Pb, Cb, h1, x, Wl, bl, Wr, gamma, beta, block_rows):
    n, d = x.shape
    wa = Pa.shape[2]
    wcb = Cb.shape[2]
    dcb = d - wa

    def body(pa_ref, pb_ref, cb_ref, h_ref, x_ref, wl_ref, bl_ref, wr_ref,
             g_ref, b_ref, o_ref):
        agg = jnp.concatenate(
            [pa_ref[0] + pa_ref[1], pb_ref[0] + pb_ref[1]], axis=1)
        cnt = jnp.mean(cb_ref[0, :, dcb:] + cb_ref[1, :, dcb:], axis=1,
                       keepdims=True)
        mean = agg / jnp.maximum(cnt, 1.0)
        h = jnp.dot(mean, wl_ref[...], preferred_element_type=jnp.float32)
        h = h + bl_ref[...]
        h = h + jnp.dot(h_ref[...], wr_ref[...], preferred_element_type=jnp.float32)
        h = h + x_ref[...]
        mu = jnp.mean(h, axis=1, keepdims=True)
        hc = h - mu
        var = jnp.mean(hc * hc, axis=1, keepdims=True)
        o_ref[...] = hc * lax.rsqrt(var + 1e-5) * g_ref[...] + b_ref[...]

    return pl.pallas_call(
        body,
        grid=(n // block_rows,),
        in_specs=[
            pl.BlockSpec((_NC, block_rows, wa), lambda i: (0, i, 0)),
            pl.BlockSpec((_NC, block_rows, d - wa), lambda i: (0, i, 0)),
            pl.BlockSpec((_NC, block_rows, wcb), lambda i: (0, i, 0)),
            pl.BlockSpec((block_rows, d), lambda i: (i, 0)),
            pl.BlockSpec((block_rows, d), lambda i: (i, 0)),
            pl.BlockSpec((d, d), lambda i: (0, 0)),
            pl.BlockSpec((1, d), lambda i: (0, 0)),
            pl.BlockSpec((d, d), lambda i: (0, 0)),
            pl.BlockSpec((1, d), lambda i: (0, 0)),
            pl.BlockSpec((1, d), lambda i: (0, 0)),
        ],
        out_specs=pl.BlockSpec((block_rows, d), lambda i: (i, 0)),
        out_shape=jax.ShapeDtypeStruct((n, d), jnp.float32),
    )(Pa, Pb, Cb, h1, x, Wl, bl.reshape(1, d), Wr, gamma.reshape(1, d),
      beta.reshape(1, d))


def kernel(x, edge_index, W1l, b1l, W1r, W2l, b2l, W2r, gamma, beta):
    n, d = x.shape
    e = edge_index.shape[1]

    e_pad = _round_up(e, _NW * _CHUNK * _G * 2)
    n_pad = _round_up(n + 1, 16 * _NS)

    src = edge_index[0]
    dst = edge_index[1]
    if e_pad != e:
        pad = e_pad - e
        src = jnp.concatenate([src, jnp.zeros((pad,), jnp.int32)])
        dst = jnp.concatenate([dst, jnp.full((pad,), n, jnp.int32)])
    nchunk = e_pad // (_NW * _CHUNK)
    e2 = jnp.stack([src.reshape(_NW, nchunk, _CHUNK),
                    dst.reshape(_NW, nchunk, _CHUNK)], axis=2)

    w1 = d + _CW
    ha = (w1 + 1) // 2
    xa = x[:, :ha]
    xb = jnp.concatenate([x[:, ha:], jnp.ones((n, _CW), jnp.float32)], axis=1)

    P1a = _make_sc_agg(n, n_pad, ha, e_pad)(e2, xa)
    P1b = _make_sc_agg(n, n_pad, w1 - ha, e_pad)(e2, xb)
    h1 = _tc_layer1(P1a, P1b, x, W1l, b1l, W1r, 400)
    hh = d // 2
    P2a = _make_sc_agg(n, n_pad, hh, e_pad)(e2, h1[:, :hh])
    P2b = _make_sc_agg(n, n_pad, hh, e_pad)(e2, h1[:, hh:])
    return _tc_layer2(P2a, P2b, P1b, h1, x, W2l, b2l, W2r, gamma, beta, 400)


# speedup vs baseline: 2.0725x; 1.0586x over previous
import functools

import jax
import jax.numpy as jnp
from jax import lax
from jax.experimental import pallas as pl
from jax.experimental.pallas import tpu as pltpu
from jax.experimental.pallas import tpu_sc as plsc

_NC = 2
_NS = 16
_CHUNK = 128
_G = 8
_CW = 16


def _round_up(a: int, b: int) -> int:
    return (a + b - 1) // b * b


@functools.lru_cache(maxsize=None)
def _make_sc_agg(n: int, n_pad: int, w: int, e_pad: int):
    eps = e_pad // _NS
    nchunk = eps // _CHUNK
    ngroups = nchunk // _G
    assert ngroups % 2 == 0
    rows_ps = n_pad // _NS
    assert rows_ps % 16 == 0
    assert n % _NS == 0
    tbl_ps = n // _NS

    mesh = plsc.VectorSubcoreMesh(core_axis_name="c", subcore_axis_name="s")
    scratch = [
        pltpu.VMEM((_G, 2, _CHUNK), jnp.int32),
        pltpu.VMEM((_G, 2, _CHUNK), jnp.int32),
        pltpu.VMEM((_CHUNK, w), jnp.float32),
        pltpu.VMEM((_CHUNK, w), jnp.float32),
        pltpu.VMEM((16, w), jnp.float32),
        pltpu.VMEM_SHARED((n, w), jnp.float32),
        pltpu.VMEM_SHARED((n_pad, w), jnp.float32),
        pltpu.SemaphoreType.DMA,
        pltpu.SemaphoreType.DMA,
        pltpu.SemaphoreType.DMA,
        pltpu.SemaphoreType.DMA,
        pltpu.SemaphoreType.DMA,
        pltpu.SemaphoreType.DMA,
    ]

    def body(e2_h, h_h, acc_o, iga, igb, r0, r1, zbuf, tbl, acc,
             sia, sib, sg0, sg1, ss0, ss1):
        c = lax.axis_index("c")
        s = lax.axis_index("s")

        pltpu.sync_copy(h_h.at[c, pl.ds(s * tbl_ps, tbl_ps)],
                        tbl.at[pl.ds(s * tbl_ps, tbl_ps)])

        z16 = jnp.zeros((16,), jnp.float32)
        for i in range(16):
            for j in range(w // 16):
                zbuf[i, pl.ds(j * 16, 16)] = z16

        base_row = s * rows_ps

        def zero_body(t, carry):
            pltpu.sync_copy(zbuf, acc.at[pl.ds(base_row + t * 16, 16)])
            return carry

        lax.fori_loop(0, rows_ps // 16, zero_body, 0)
        pltpu.sync_copy(e2_h.at[s, pl.ds(0, _G)], iga)
        plsc.subcore_barrier()

        def process_group(ig):
            dg = [None, None]
            dg[0] = pltpu.async_copy(tbl.at[ig.at[0, 0]], r0, sg0)
            for p in range(_G // 2):
                j0 = 2 * p
                dg[0].wait()
                ds0 = pltpu.async_copy(r0, acc.at[ig.at[j0, 1]], ss0, add=True)
                dg[1] = pltpu.async_copy(tbl.at[ig.at[j0 + 1, 0]], r1, sg1)
                ds0.wait()
                if j0 + 2 < _G:
                    dg[0] = pltpu.async_copy(tbl.at[ig.at[j0 + 2, 0]], r0, sg0)
                dg[1].wait()
                ds1 = pltpu.async_copy(r1, acc.at[ig.at[j0 + 1, 1]], ss1,
                                       add=True)
                ds1.wait()

        def outer_body(t, carry):
            ga = 2 * t
            dpb = pltpu.async_copy(e2_h.at[s, pl.ds((ga + 1) * _G, _G)],
                                   igb, sib)
            process_group(iga)
            dpb.wait()
            dpa = pltpu.async_copy(
                e2_h.at[s, pl.ds(lax.rem((ga + 2), ngroups) * _G, _G)],
                iga, sia)
            process_group(igb)
            dpa.wait()
            return carry

        lax.fori_loop(0, ngroups // 2, outer_body, 0)
        plsc.subcore_barrier()

        pltpu.sync_copy(acc.at[pl.ds(base_row, rows_ps)],
                        acc_o.at[c, pl.ds(base_row, rows_ps)])

    return pl.kernel(
        body,
        out_type=jax.ShapeDtypeStruct((_NC, n_pad, w), jnp.float32),
        mesh=mesh, scratch_types=scratch,
        compiler_params=pltpu.CompilerParams(use_tc_tiling_on_sc=False))


def _tc_layer1(P, x, Wl, bl, Wr, block_rows):
    n, d = x.shape
    wa = P.shape[2]
    db = d - wa

    def body(p_ref, x_ref, wl_ref, bl_ref, wr_ref, o_ref):
        agg = jnp.concatenate([p_ref[0], p_ref[1, :, :db]], axis=1)
        cnt = jnp.mean(p_ref[1, :, db:], axis=1, keepdims=True)
        mean = agg / jnp.maximum(cnt, 1.0)
        h = jnp.dot(mean, wl_ref[...], preferred_element_type=jnp.float32)
        h = h + bl_ref[...]
        h = h + jnp.dot(x_ref[...], wr_ref[...], preferred_element_type=jnp.float32)
        o_ref[...] = jnp.maximum(h, 0.0)

    return pl.pallas_call(
        body,
        grid=(n // block_rows,),
        in_specs=[
            pl.BlockSpec((_NC, block_rows, wa), lambda i: (0, i, 0)),
            pl.BlockSpec((block_rows, d), lambda i: (i, 0)),
            pl.BlockSpec((d, d), lambda i: (0, 0)),
            pl.BlockSpec((1, d), lambda i: (0, 0)),
            pl.BlockSpec((d, d), lambda i: (0, 0)),
        ],
        out_specs=pl.BlockSpec((block_rows, d), lambda i: (i, 0)),
        out_shape=jax.ShapeDtypeStruct((n, d), jnp.float32),
    )(P, x, Wl, bl.reshape(1, d), Wr)


def _tc_layer2(P, C1, h1, x, Wl, bl, Wr, gamma, beta, block_rows):
    n, d = x.shape
    wc = C1.shape[2]
    db = d - wc

    def body(p_ref, c_ref, h_ref, x_ref, wl_ref, bl_ref, wr_ref,
             g_ref, b_ref, o_ref):
        agg = jnp.concatenate([p_ref[0], p_ref[1]], axis=1)
        cnt = jnp.mean(c_ref[0, :, db:], axis=1, keepdims=True)
        mean = agg / jnp.maximum(cnt, 1.0)
        h = jnp.dot(mean, wl_ref[...], preferred_element_type=jnp.float32)
        h = h + bl_ref[...]
        h = h + jnp.dot(h_ref[...], wr_ref[...], preferred_element_type=jnp.float32)
        h = h + x_ref[...]
        mu = jnp.mean(h, axis=1, keepdims=True)
        hc = h - mu
        var = jnp.mean(hc * hc, axis=1, keepdims=True)
        o_ref[...] = hc * lax.rsqrt(var + 1e-5) * g_ref[...] + b_ref[...]

    return pl.pallas_call(
        body,
        grid=(n // block_rows,),
        in_specs=[
            pl.BlockSpec((_NC, block_rows, d // 2), lambda i: (0, i, 0)),
            pl.BlockSpec((1, block_rows, wc), lambda i: (1, i, 0)),
            pl.BlockSpec((block_rows, d), lambda i: (i, 0)),
            pl.BlockSpec((block_rows, d), lambda i: (i, 0)),
            pl.BlockSpec((d, d), lambda i: (0, 0)),
            pl.BlockSpec((1, d), lambda i: (0, 0)),
            pl.BlockSpec((d, d), lambda i: (0, 0)),
            pl.BlockSpec((1, d), lambda i: (0, 0)),
            pl.BlockSpec((1, d), lambda i: (0, 0)),
        ],
        out_specs=pl.BlockSpec((block_rows, d), lambda i: (i, 0)),
        out_shape=jax.ShapeDtypeStruct((n, d), jnp.float32),
    )(P, C1, h1, x, Wl, bl.reshape(1, d), Wr, gamma.reshape(1, d),
      beta.reshape(1, d))


def kernel(x, edge_index, W1l, b1l, W1r, W2l, b2l, W2r, gamma, beta):
    n, d = x.shape
    e = edge_index.shape[1]

    e_pad = _round_up(e, _NS * _CHUNK * _G * 2)
    n_pad = _round_up(n + 1, 16 * _NS)

    src = edge_index[0]
    dst = edge_index[1]
    if e_pad != e:
        pad = e_pad - e
        src = jnp.concatenate([src, jnp.zeros((pad,), jnp.int32)])
        dst = jnp.concatenate([dst, jnp.full((pad,), n, jnp.int32)])
    nchunk = e_pad // (_NS * _CHUNK)
    e2 = jnp.stack([src.reshape(_NS, nchunk, _CHUNK),
                    dst.reshape(_NS, nchunk, _CHUNK)], axis=2)

    w1 = d + _CW
    ha = w1 // 2
    x2 = jnp.stack([x[:, :ha],
                    jnp.concatenate([x[:, ha:],
                                     jnp.ones((n, _CW), jnp.float32)], axis=1)])

    P1 = _make_sc_agg(n, n_pad, ha, e_pad)(e2, x2)
    h1 = _tc_layer1(P1, x, W1l, b1l, W1r, 400)
    hh = d // 2
    h2 = jnp.stack([h1[:, :hh], h1[:, hh:]])
    P2 = _make_sc_agg(n, n_pad, hh, e_pad)(e2, h2)
    return _tc_layer2(P2, P1, h1, x, W2l, b2l, W2r, gamma, beta, 400)


# speedup vs baseline: 2.3388x; 1.1285x over previous
import functools

import jax
import jax.numpy as jnp
from jax import lax
from jax.experimental import pallas as pl
from jax.experimental.pallas import tpu as pltpu
from jax.experimental.pallas import tpu_sc as plsc

_NC = 2
_NS = 16
_CHUNK = 128
_G = 16
_CW = 16


def _round_up(a: int, b: int) -> int:
    return (a + b - 1) // b * b


@functools.lru_cache(maxsize=None)
def _make_sc_agg(n: int, n_pad: int, w: int, e_pad: int):
    eps = e_pad // _NS
    nchunk = eps // _CHUNK
    ngroups = nchunk // _G
    assert ngroups % 2 == 0
    rows_ps = n_pad // _NS
    assert rows_ps % 16 == 0
    assert n % _NS == 0
    tbl_ps = n // _NS

    mesh = plsc.VectorSubcoreMesh(core_axis_name="c", subcore_axis_name="s")
    scratch = [
        pltpu.VMEM((_G, 2, _CHUNK), jnp.int32),
        pltpu.VMEM((_G, 2, _CHUNK), jnp.int32),
        pltpu.VMEM((_CHUNK, w), jnp.float32),
        pltpu.VMEM((_CHUNK, w), jnp.float32),
        pltpu.VMEM((_CHUNK, w), jnp.float32),
        pltpu.VMEM((16, w), jnp.float32),
        pltpu.VMEM_SHARED((n, w), jnp.float32),
        pltpu.VMEM_SHARED((n_pad, w), jnp.float32),
        pltpu.SemaphoreType.DMA,
        pltpu.SemaphoreType.DMA,
        pltpu.SemaphoreType.DMA,
        pltpu.SemaphoreType.DMA,
        pltpu.SemaphoreType.DMA,
        pltpu.SemaphoreType.DMA,
        pltpu.SemaphoreType.DMA,
        pltpu.SemaphoreType.DMA,
    ]

    def body(e2_h, h_h, acc_o, iga, igb, r0, r1, r2, zbuf, tbl, acc,
             sia, sib, sg0, sg1, sg2, ss0, ss1, ss2):
        c = lax.axis_index("c")
        s = lax.axis_index("s")

        pltpu.sync_copy(h_h.at[c, pl.ds(s * tbl_ps, tbl_ps)],
                        tbl.at[pl.ds(s * tbl_ps, tbl_ps)])

        z16 = jnp.zeros((16,), jnp.float32)
        for i in range(16):
            for j in range(w // 16):
                zbuf[i, pl.ds(j * 16, 16)] = z16

        base_row = s * rows_ps

        def zero_body(t, carry):
            pltpu.sync_copy(zbuf, acc.at[pl.ds(base_row + t * 16, 16)])
            return carry

        lax.fori_loop(0, rows_ps // 16, zero_body, 0)
        pltpu.sync_copy(e2_h.at[s, pl.ds(0, _G)], iga)
        plsc.subcore_barrier()

        rb = (r0, r1, r2)
        sg = (sg0, sg1, sg2)
        ss = (ss0, ss1, ss2)

        def process_group(ig):
            dg = [None, None, None]
            ds = [None, None, None]
            dg[0] = pltpu.async_copy(tbl.at[ig.at[0, 0]], rb[0], sg[0])
            dg[1] = pltpu.async_copy(tbl.at[ig.at[1, 0]], rb[1], sg[1])
            for j in range(_G):
                b = j % 3
                dg[b].wait()
                ds[b] = pltpu.async_copy(rb[b], acc.at[ig.at[j, 1]], ss[b],
                                         add=True)
                if j + 2 < _G:
                    b2 = (j + 2) % 3
                    if ds[b2] is not None:
                        ds[b2].wait()
                    dg[b2] = pltpu.async_copy(tbl.at[ig.at[j + 2, 0]],
                                              rb[b2], sg[b2])
            for b in ((_G - 3) % 3, (_G - 2) % 3, (_G - 1) % 3):
                ds[b].wait()

        def outer_body(t, carry):
            ga = 2 * t
            dpb = pltpu.async_copy(e2_h.at[s, pl.ds((ga + 1) * _G, _G)],
                                   igb, sib)
            process_group(iga)
            dpb.wait()
            dpa = pltpu.async_copy(
                e2_h.at[s, pl.ds(lax.rem((ga + 2), ngroups) * _G, _G)],
                iga, sia)
            process_group(igb)
            dpa.wait()
            return carry

        lax.fori_loop(0, ngroups // 2, outer_body, 0)
        plsc.subcore_barrier()

        pltpu.sync_copy(acc.at[pl.ds(base_row, rows_ps)],
                        acc_o.at[c, pl.ds(base_row, rows_ps)])

    return pl.kernel(
        body,
        out_type=jax.ShapeDtypeStruct((_NC, n_pad, w), jnp.float32),
        mesh=mesh, scratch_types=scratch,
        compiler_params=pltpu.CompilerParams(use_tc_tiling_on_sc=False))


def _tc_layer1(P, x, Wl, bl, Wr, block_rows):
    n, d = x.shape
    wa = P.shape[2]
    db = d - wa

    def body(p_ref, x_ref, wl_ref, bl_ref, wr_ref, o_ref):
        agg = jnp.concatenate([p_ref[0], p_ref[1, :, :db]], axis=1)
        cnt = jnp.mean(p_ref[1, :, db:], axis=1, keepdims=True)
        mean = agg / jnp.maximum(cnt, 1.0)
        h = jnp.dot(mean, wl_ref[...], preferred_element_type=jnp.float32)
        h = h + bl_ref[...]
        h = h + jnp.dot(x_ref[...], wr_ref[...], preferred_element_type=jnp.float32)
        o_ref[...] = jnp.maximum(h, 0.0)

    return pl.pallas_call(
        body,
        grid=(n // block_rows,),
        in_specs=[
            pl.BlockSpec((_NC, block_rows, wa), lambda i: (0, i, 0)),
            pl.BlockSpec((block_rows, d), lambda i: (i, 0)),
            pl.BlockSpec((d, d), lambda i: (0, 0)),
            pl.BlockSpec((1, d), lambda i: (0, 0)),
            pl.BlockSpec((d, d), lambda i: (0, 0)),
        ],
        out_specs=pl.BlockSpec((block_rows, d), lambda i: (i, 0)),
        out_shape=jax.ShapeDtypeStruct((n, d), jnp.float32),
    )(P, x, Wl, bl.reshape(1, d), Wr)


def _tc_layer2(P, C1, h1, x, Wl, bl, Wr, gamma, beta, block_rows):
    n, d = x.shape
    wc = C1.shape[2]
    db = d - wc

    def body(p_ref, c_ref, h_ref, x_ref, wl_ref, bl_ref, wr_ref,
             g_ref, b_ref, o_ref):
        agg = jnp.concatenate([p_ref[0], p_ref[1]], axis=1)
        cnt = jnp.mean(c_ref[0, :, db:], axis=1, keepdims=True)
        mean = agg / jnp.maximum(cnt, 1.0)
        h = jnp.dot(mean, wl_ref[...], preferred_element_type=jnp.float32)
        h = h + bl_ref[...]
        h = h + jnp.dot(h_ref[...], wr_ref[...], preferred_element_type=jnp.float32)
        h = h + x_ref[...]
        mu = jnp.mean(h, axis=1, keepdims=True)
        hc = h - mu
        var = jnp.mean(hc * hc, axis=1, keepdims=True)
        o_ref[...] = hc * lax.rsqrt(var + 1e-5) * g_ref[...] + b_ref[...]

    return pl.pallas_call(
        body,
        grid=(n // block_rows,),
        in_specs=[
            pl.BlockSpec((_NC, block_rows, d // 2), lambda i: (0, i, 0)),
            pl.BlockSpec((1, block_rows, wc), lambda i: (1, i, 0)),
            pl.BlockSpec((block_rows, d), lambda i: (i, 0)),
            pl.BlockSpec((block_rows, d), lambda i: (i, 0)),
            pl.BlockSpec((d, d), lambda i: (0, 0)),
            pl.BlockSpec((1, d), lambda i: (0, 0)),
            pl.BlockSpec((d, d), lambda i: (0, 0)),
            pl.BlockSpec((1, d), lambda i: (0, 0)),
            pl.BlockSpec((1, d), lambda i: (0, 0)),
        ],
        out_specs=pl.BlockSpec((block_rows, d), lambda i: (i, 0)),
        out_shape=jax.ShapeDtypeStruct((n, d), jnp.float32),
    )(P, C1, h1, x, Wl, bl.reshape(1, d), Wr, gamma.reshape(1, d),
      beta.reshape(1, d))


def kernel(x, edge_index, W1l, b1l, W1r, W2l, b2l, W2r, gamma, beta):
    n, d = x.shape
    e = edge_index.shape[1]

    e_pad = _round_up(e, _NS * _CHUNK * _G * 2)
    n_pad = _round_up(n + 1, 16 * _NS)

    src = edge_index[0]
    dst = edge_index[1]
    if e_pad != e:
        pad = e_pad - e
        src = jnp.concatenate([src, jnp.zeros((pad,), jnp.int32)])
        dst = jnp.concatenate([dst, jnp.full((pad,), n, jnp.int32)])
    nchunk = e_pad // (_NS * _CHUNK)
    e2 = jnp.stack([src.reshape(_NS, nchunk, _CHUNK),
                    dst.reshape(_NS, nchunk, _CHUNK)], axis=2)

    w1 = d + _CW
    ha = w1 // 2
    x2 = jnp.stack([x[:, :ha],
                    jnp.concatenate([x[:, ha:],
                                     jnp.ones((n, _CW), jnp.float32)], axis=1)])

    P1 = _make_sc_agg(n, n_pad, ha, e_pad)(e2, x2)
    h1 = _tc_layer1(P1, x, W1l, b1l, W1r, 400)
    hh = d // 2
    h2 = jnp.stack([h1[:, :hh], h1[:, hh:]])
    P2 = _make_sc_agg(n, n_pad, hh, e_pad)(e2, h2)
    return _tc_layer2(P2, P1, h1, x, W2l, b2l, W2r, gamma, beta, 400)


# speedup vs baseline: 2.3758x; 1.0158x over previous
import functools

import jax
import jax.numpy as jnp
from jax import lax
from jax.experimental import pallas as pl
from jax.experimental.pallas import tpu as pltpu
from jax.experimental.pallas import tpu_sc as plsc

_NC = 2
_NS = 16
_CHUNK = 128
_G = 16
_CW = 16


def _round_up(a: int, b: int) -> int:
    return (a + b - 1) // b * b


@functools.lru_cache(maxsize=None)
def _make_sc_agg(n: int, n_pad: int, w: int, e_pad: int):
    eps = e_pad // _NS
    nchunk = eps // _CHUNK
    ngroups = nchunk // _G
    assert ngroups % 2 == 0
    rows_ps = n_pad // _NS
    assert rows_ps % 16 == 0
    assert n % _NS == 0
    tbl_ps = n // _NS

    mesh = plsc.VectorSubcoreMesh(core_axis_name="c", subcore_axis_name="s")
    scratch = [
        pltpu.VMEM((_G, 2, _CHUNK), jnp.int32),
        pltpu.VMEM((_G, 2, _CHUNK), jnp.int32),
        pltpu.VMEM((_CHUNK, w), jnp.float32),
        pltpu.VMEM((_CHUNK, w), jnp.float32),
        pltpu.VMEM((_CHUNK, w), jnp.float32),
        pltpu.VMEM((16, w), jnp.float32),
        pltpu.VMEM_SHARED((n, w), jnp.float32),
        pltpu.VMEM_SHARED((n_pad, w), jnp.float32),
        pltpu.SemaphoreType.DMA,
        pltpu.SemaphoreType.DMA,
        pltpu.SemaphoreType.DMA,
        pltpu.SemaphoreType.DMA,
        pltpu.SemaphoreType.DMA,
        pltpu.SemaphoreType.DMA,
        pltpu.SemaphoreType.DMA,
        pltpu.SemaphoreType.DMA,
    ]

    def body(e2_h, h_h, acc_o, iga, igb, r0, r1, r2, zbuf, tbl, acc,
             sia, sib, sg0, sg1, sg2, ss0, ss1, ss2):
        c = lax.axis_index("c")
        s = lax.axis_index("s")

        pltpu.sync_copy(h_h.at[c, pl.ds(s * tbl_ps, tbl_ps)],
                        tbl.at[pl.ds(s * tbl_ps, tbl_ps)])

        z16 = jnp.zeros((16,), jnp.float32)
        for i in range(16):
            for j in range(w // 16):
                zbuf[i, pl.ds(j * 16, 16)] = z16

        base_row = s * rows_ps

        def zero_body(t, carry):
            pltpu.sync_copy(zbuf, acc.at[pl.ds(base_row + t * 16, 16)])
            return carry

        def pref(g, ig, sem):
            return pltpu.async_copy(e2_h.at[s, pl.ds(g * _G, _G)], ig, sem)

        def pref_wait(d):
            d.wait()

        lax.fori_loop(0, rows_ps // 16, zero_body, 0)
        pref_wait(pref(0, iga, sia))
        plsc.subcore_barrier()

        rb = (r0, r1, r2)
        sg = (sg0, sg1, sg2)
        ss = (ss0, ss1, ss2)

        def process_group(ig):
            dg = [None, None, None]
            ds = [None, None, None]
            dg[0] = pltpu.async_copy(tbl.at[ig.at[0, 0]], rb[0], sg[0])
            dg[1] = pltpu.async_copy(tbl.at[ig.at[1, 0]], rb[1], sg[1])
            for j in range(_G):
                b = j % 3
                dg[b].wait()
                ds[b] = pltpu.async_copy(rb[b], acc.at[ig.at[j, 1]], ss[b],
                                         add=True)
                if j + 2 < _G:
                    b2 = (j + 2) % 3
                    if ds[b2] is not None:
                        ds[b2].wait()
                    dg[b2] = pltpu.async_copy(tbl.at[ig.at[j + 2, 0]],
                                              rb[b2], sg[b2])
            for b in ((_G - 3) % 3, (_G - 2) % 3, (_G - 1) % 3):
                ds[b].wait()

        def outer_body(t, carry):
            ga = 2 * t
            dpb = pref(ga + 1, igb, sib)
            process_group(iga)
            pref_wait(dpb)
            dpa = pref(lax.rem(ga + 2, ngroups), iga, sia)
            process_group(igb)
            pref_wait(dpa)
            return carry

        lax.fori_loop(0, ngroups // 2, outer_body, 0)
        plsc.subcore_barrier()

        pltpu.sync_copy(acc.at[pl.ds(base_row, rows_ps)],
                        acc_o.at[c, pl.ds(base_row, rows_ps)])

    return pl.kernel(
        body,
        out_type=jax.ShapeDtypeStruct((_NC, n_pad, w), jnp.float32),
        mesh=mesh, scratch_types=scratch,
        compiler_params=pltpu.CompilerParams(use_tc_tiling_on_sc=False))


def _tc_layer1(P, x, Wl, bl, Wr, block_rows):
    n, d = x.shape
    wa = P.shape[2]
    db = d - wa
    hh = d // 2

    def body(p_ref, x_ref, wl_ref, bl_ref, wr_ref, o_ref):
        agg = jnp.concatenate([p_ref[0], p_ref[1, :, :db]], axis=1)
        cnt = jnp.mean(p_ref[1, :, db:], axis=1, keepdims=True)
        mean = agg / jnp.maximum(cnt, 1.0)
        h = jnp.dot(mean, wl_ref[...], preferred_element_type=jnp.float32)
        h = h + bl_ref[...]
        h = h + jnp.dot(x_ref[...], wr_ref[...], preferred_element_type=jnp.float32)
        h = jnp.maximum(h, 0.0)
        o_ref[0] = h[:, :hh]
        o_ref[1] = h[:, hh:]

    return pl.pallas_call(
        body,
        grid=(n // block_rows,),
        in_specs=[
            pl.BlockSpec((_NC, block_rows, wa), lambda i: (0, i, 0)),
            pl.BlockSpec((block_rows, d), lambda i: (i, 0)),
            pl.BlockSpec((d, d), lambda i: (0, 0)),
            pl.BlockSpec((1, d), lambda i: (0, 0)),
            pl.BlockSpec((d, d), lambda i: (0, 0)),
        ],
        out_specs=pl.BlockSpec((2, block_rows, hh), lambda i: (0, i, 0)),
        out_shape=jax.ShapeDtypeStruct((2, n, hh), jnp.float32),
    )(P, x, Wl, bl.reshape(1, d), Wr)


def _tc_layer2(P, C1, h2, x, Wl, bl, Wr, gamma, beta, block_rows):
    n, d = x.shape
    wc = C1.shape[2]
    db = d - wc
    hh = d // 2

    def body(p_ref, c_ref, h_ref, x_ref, wl_ref, bl_ref, wr_ref,
             g_ref, b_ref, o_ref):
        agg = jnp.concatenate([p_ref[0], p_ref[1]], axis=1)
        cnt = jnp.mean(c_ref[0, :, db:], axis=1, keepdims=True)
        mean = agg / jnp.maximum(cnt, 1.0)
        h = jnp.dot(mean, wl_ref[...], preferred_element_type=jnp.float32)
        h = h + bl_ref[...]
        h = h + jnp.dot(h_ref[0], wr_ref[0], preferred_element_type=jnp.float32)
        h = h + jnp.dot(h_ref[1], wr_ref[1], preferred_element_type=jnp.float32)
        h = h + x_ref[...]
        mu = jnp.mean(h, axis=1, keepdims=True)
        hc = h - mu
        var = jnp.mean(hc * hc, axis=1, keepdims=True)
        o_ref[...] = hc * lax.rsqrt(var + 1e-5) * g_ref[...] + b_ref[...]

    return pl.pallas_call(
        body,
        grid=(n // block_rows,),
        in_specs=[
            pl.BlockSpec((_NC, block_rows, hh), lambda i: (0, i, 0)),
            pl.BlockSpec((1, block_rows, wc), lambda i: (1, i, 0)),
            pl.BlockSpec((2, block_rows, hh), lambda i: (0, i, 0)),
            pl.BlockSpec((block_rows, d), lambda i: (i, 0)),
            pl.BlockSpec((d, d), lambda i: (0, 0)),
            pl.BlockSpec((1, d), lambda i: (0, 0)),
            pl.BlockSpec((2, hh, d), lambda i: (0, 0, 0)),
            pl.BlockSpec((1, d), lambda i: (0, 0)),
            pl.BlockSpec((1, d), lambda i: (0, 0)),
        ],
        out_specs=pl.BlockSpec((block_rows, d), lambda i: (i, 0)),
        out_shape=jax.ShapeDtypeStruct((n, d), jnp.float32),
    )(P, C1, h2, x, Wl, bl.reshape(1, d), Wr.reshape(2, hh, d),
      gamma.reshape(1, d), beta.reshape(1, d))


def kernel(x, edge_index, W1l, b1l, W1r, W2l, b2l, W2r, gamma, beta):
    n, d = x.shape
    e = edge_index.shape[1]

    e_pad = _round_up(e, _NS * _CHUNK * _G * 2)
    n_pad = _round_up(n + 1, 16 * _NS)

    src = edge_index[0]
    dst = edge_index[1]
    if e_pad != e:
        pad = e_pad - e
        src = jnp.concatenate([src, jnp.zeros((pad,), jnp.int32)])
        dst = jnp.concatenate([dst, jnp.full((pad,), n, jnp.int32)])
    nchunk = e_pad // (_NS * _CHUNK)
    e2 = jnp.stack([src.reshape(_NS, nchunk, _CHUNK),
                    dst.reshape(_NS, nchunk, _CHUNK)], axis=2)

    w1 = d + _CW
    ha = w1 // 2
    x2 = jnp.stack([x[:, :ha],
                    jnp.concatenate([x[:, ha:],
                                     jnp.ones((n, _CW), jnp.float32)], axis=1)])

    P1 = _make_sc_agg(n, n_pad, ha, e_pad)(e2, x2)
    h2 = _tc_layer1(P1, x, W1l, b1l, W1r, 400)
    P2 = _make_sc_agg(n, n_pad, d // 2, e_pad)(e2, h2)
    return _tc_layer2(P2, P1, h2, x, W2l, b2l, W2r, gamma, beta, 400)


# speedup vs baseline: 2.4194x; 1.0184x over previous
import functools

import jax
import jax.numpy as jnp
from jax import lax
from jax.experimental import pallas as pl
from jax.experimental.pallas import tpu as pltpu
from jax.experimental.pallas import tpu_sc as plsc

_NC = 2
_NS = 16
_CHUNK = 128
_G = 20
_CW = 16


def _round_up(a: int, b: int) -> int:
    return (a + b - 1) // b * b


@functools.lru_cache(maxsize=None)
def _make_sc_agg(n: int, n_pad: int, w: int, e_pad: int):
    eps = e_pad // _NS
    nchunk = eps // _CHUNK
    ngroups = nchunk // _G
    assert ngroups % 2 == 0
    rows_ps = n_pad // _NS
    assert rows_ps % 16 == 0
    assert n % _NS == 0
    tbl_ps = n // _NS

    mesh = plsc.VectorSubcoreMesh(core_axis_name="c", subcore_axis_name="s")
    scratch = [
        pltpu.VMEM((_G, 2, _CHUNK), jnp.int32),
        pltpu.VMEM((_G, 2, _CHUNK), jnp.int32),
        pltpu.VMEM((_CHUNK, w), jnp.float32),
        pltpu.VMEM((_CHUNK, w), jnp.float32),
        pltpu.VMEM((_CHUNK, w), jnp.float32),
        pltpu.VMEM((16, w), jnp.float32),
        pltpu.VMEM_SHARED((n, w), jnp.float32),
        pltpu.VMEM_SHARED((n_pad, w), jnp.float32),
        pltpu.SemaphoreType.DMA,
        pltpu.SemaphoreType.DMA,
        pltpu.SemaphoreType.DMA,
        pltpu.SemaphoreType.DMA,
        pltpu.SemaphoreType.DMA,
        pltpu.SemaphoreType.DMA,
        pltpu.SemaphoreType.DMA,
        pltpu.SemaphoreType.DMA,
    ]

    def body(e2_h, h_h, acc_o, iga, igb, r0, r1, r2, zbuf, tbl, acc,
             sia, sib, sg0, sg1, sg2, ss0, ss1, ss2):
        c = lax.axis_index("c")
        s = lax.axis_index("s")

        pltpu.sync_copy(h_h.at[c, pl.ds(s * tbl_ps, tbl_ps)],
                        tbl.at[pl.ds(s * tbl_ps, tbl_ps)])

        z16 = jnp.zeros((16,), jnp.float32)
        for i in range(16):
            for j in range(w // 16):
                zbuf[i, pl.ds(j * 16, 16)] = z16

        base_row = s * rows_ps

        def zero_body(t, carry):
            pltpu.sync_copy(zbuf, acc.at[pl.ds(base_row + t * 16, 16)])
            return carry

        def pref(g, ig, sem):
            return pltpu.async_copy(e2_h.at[s, pl.ds(g * _G, _G)], ig, sem)

        def pref_wait(d):
            d.wait()

        lax.fori_loop(0, rows_ps // 16, zero_body, 0)
        pref_wait(pref(0, iga, sia))
        plsc.subcore_barrier()

        rb = (r0, r1, r2)
        sg = (sg0, sg1, sg2)
        ss = (ss0, ss1, ss2)

        def process_group(ig):
            dg = [None, None, None]
            ds = [None, None, None]
            dg[0] = pltpu.async_copy(tbl.at[ig.at[0, 0]], rb[0], sg[0])
            dg[1] = pltpu.async_copy(tbl.at[ig.at[1, 0]], rb[1], sg[1])
            for j in range(_G):
                b = j % 3
                dg[b].wait()
                ds[b] = pltpu.async_copy(rb[b], acc.at[ig.at[j, 1]], ss[b],
                                         add=True)
                if j + 2 < _G:
                    b2 = (j + 2) % 3
                    if ds[b2] is not None:
                        ds[b2].wait()
                    dg[b2] = pltpu.async_copy(tbl.at[ig.at[j + 2, 0]],
                                              rb[b2], sg[b2])
            for b in ((_G - 3) % 3, (_G - 2) % 3, (_G - 1) % 3):
                ds[b].wait()

        def outer_body(t, carry):
            ga = 2 * t
            dpb = pref(ga + 1, igb, sib)
            process_group(iga)
            pref_wait(dpb)
            dpa = pref(lax.rem(ga + 2, ngroups), iga, sia)
            process_group(igb)
            pref_wait(dpa)
            return carry

        lax.fori_loop(0, ngroups // 2, outer_body, 0)
        plsc.subcore_barrier()

        pltpu.sync_copy(acc.at[pl.ds(base_row, rows_ps)],
                        acc_o.at[c, pl.ds(base_row, rows_ps)])

    return pl.kernel(
        body,
        out_type=jax.ShapeDtypeStruct((_NC, n_pad, w), jnp.float32),
        mesh=mesh, scratch_types=scratch,
        compiler_params=pltpu.CompilerParams(use_tc_tiling_on_sc=False))


def _tc_layer1(P, x, Wl, bl, Wr, block_rows):
    n, d = x.shape
    wa = P.shape[2]
    db = d - wa
    hh = d // 2

    def body(p_ref, x_ref, wl_ref, bl_ref, wr_ref, o_ref):
        agg = jnp.concatenate([p_ref[0], p_ref[1, :, :db]], axis=1)
        cnt = jnp.mean(p_ref[1, :, db:], axis=1, keepdims=True)
        mean = agg / jnp.maximum(cnt, 1.0)
        h = jnp.dot(mean, wl_ref[...], preferred_element_type=jnp.float32)
        h = h + bl_ref[...]
        h = h + jnp.dot(x_ref[...], wr_ref[...], preferred_element_type=jnp.float32)
        h = jnp.maximum(h, 0.0)
        o_ref[0] = h[:, :hh]
        o_ref[1] = h[:, hh:]

    return pl.pallas_call(
        body,
        grid=(n // block_rows,),
        in_specs=[
            pl.BlockSpec((_NC, block_rows, wa), lambda i: (0, i, 0)),
            pl.BlockSpec((block_rows, d), lambda i: (i, 0)),
            pl.BlockSpec((d, d), lambda i: (0, 0)),
            pl.BlockSpec((1, d), lambda i: (0, 0)),
            pl.BlockSpec((d, d), lambda i: (0, 0)),
        ],
        out_specs=pl.BlockSpec((2, block_rows, hh), lambda i: (0, i, 0)),
        out_shape=jax.ShapeDtypeStruct((2, n, hh), jnp.float32),
    )(P, x, Wl, bl.reshape(1, d), Wr)


def _tc_layer2(P, C1, h2, x, Wl, bl, Wr, gamma, beta, block_rows):
    n, d = x.shape
    wc = C1.shape[2]
    db = d - wc
    hh = d // 2

    def body(p_ref, c_ref, h_ref, x_ref, wl_ref, bl_ref, wr_ref,
             g_ref, b_ref, o_ref):
        agg = jnp.concatenate([p_ref[0], p_ref[1]], axis=1)
        cnt = jnp.mean(c_ref[0, :, db:], axis=1, keepdims=True)
        mean = agg / jnp.maximum(cnt, 1.0)
        h = jnp.dot(mean, wl_ref[...], preferred_element_type=jnp.float32)
        h = h + bl_ref[...]
        h = h + jnp.dot(h_ref[0], wr_ref[0], preferred_element_type=jnp.float32)
        h = h + jnp.dot(h_ref[1], wr_ref[1], preferred_element_type=jnp.float32)
        h = h + x_ref[...]
        mu = jnp.mean(h, axis=1, keepdims=True)
        hc = h - mu
        var = jnp.mean(hc * hc, axis=1, keepdims=True)
        o_ref[...] = hc * lax.rsqrt(var + 1e-5) * g_ref[...] + b_ref[...]

    return pl.pallas_call(
        body,
        grid=(n // block_rows,),
        in_specs=[
            pl.BlockSpec((_NC, block_rows, hh), lambda i: (0, i, 0)),
            pl.BlockSpec((1, block_rows, wc), lambda i: (1, i, 0)),
            pl.BlockSpec((2, block_rows, hh), lambda i: (0, i, 0)),
            pl.BlockSpec((block_rows, d), lambda i: (i, 0)),
            pl.BlockSpec((d, d), lambda i: (0, 0)),
            pl.BlockSpec((1, d), lambda i: (0, 0)),
            pl.BlockSpec((2, hh, d), lambda i: (0, 0, 0)),
            pl.BlockSpec((1, d), lambda i: (0, 0)),
            pl.BlockSpec((1, d), lambda i: (0, 0)),
        ],
        out_specs=pl.BlockSpec((block_rows, d), lambda i: (i, 0)),
        out_shape=jax.ShapeDtypeStruct((n, d), jnp.float32),
    )(P, C1, h2, x, Wl, bl.reshape(1, d), Wr.reshape(2, hh, d),
      gamma.reshape(1, d), beta.reshape(1, d))


def kernel(x, edge_index, W1l, b1l, W1r, W2l, b2l, W2r, gamma, beta):
    n, d = x.shape
    e = edge_index.shape[1]

    e_pad = _round_up(e, _NS * _CHUNK * _G * 2)
    n_pad = _round_up(n + 1, 16 * _NS)

    src = edge_index[0]
    dst = edge_index[1]
    if e_pad != e:
        pad = e_pad - e
        src = jnp.concatenate([src, jnp.zeros((pad,), jnp.int32)])
        dst = jnp.concatenate([dst, jnp.full((pad,), n, jnp.int32)])
    nchunk = e_pad // (_NS * _CHUNK)
    e2 = jnp.stack([src.reshape(_NS, nchunk, _CHUNK),
                    dst.reshape(_NS, nchunk, _CHUNK)], axis=2)

    w1 = d + _CW
    ha = w1 // 2
    x2 = jnp.stack([x[:, :ha],
                    jnp.concatenate([x[:, ha:],
                                     jnp.ones((n, _CW), jnp.float32)], axis=1)])

    P1 = _make_sc_agg(n, n_pad, ha, e_pad)(e2, x2)
    h2 = _tc_layer1(P1, x, W1l, b1l, W1r, 400)
    P2 = _make_sc_agg(n, n_pad, d // 2, e_pad)(e2, h2)
    return _tc_layer2(P2, P1, h2, x, W2l, b2l, W2r, gamma, beta, 400)


# speedup vs baseline: 2.4757x; 1.0233x over previous
import functools

import jax
import jax.numpy as jnp
from jax import lax
from jax.experimental import pallas as pl
from jax.experimental.pallas import tpu as pltpu
from jax.experimental.pallas import tpu_sc as plsc

_NC = 2
_NS = 16
_CHUNK = 128
_G = 20
_CW = 16


def _round_up(a: int, b: int) -> int:
    return (a + b - 1) // b * b


@functools.lru_cache(maxsize=None)
def _make_sc_agg(n: int, n_pad: int, w: int, e_pad: int):
    eps = e_pad // _NS
    nchunk = eps // _CHUNK
    ngroups = nchunk // _G
    assert ngroups % 2 == 0
    rows_ps = n_pad // _NS
    assert rows_ps % 16 == 0
    assert n % _NS == 0
    tbl_ps = n // _NS

    mesh = plsc.VectorSubcoreMesh(core_axis_name="c", subcore_axis_name="s")
    scratch = [
        pltpu.VMEM((_G, 2, _CHUNK), jnp.int32),
        pltpu.VMEM((_G, 2, _CHUNK), jnp.int32),
        pltpu.VMEM((_CHUNK, w), jnp.float32),
        pltpu.VMEM((_CHUNK, w), jnp.float32),
        pltpu.VMEM((_CHUNK, w), jnp.float32),
        pltpu.VMEM((16, w), jnp.float32),
        pltpu.VMEM_SHARED((n, w), jnp.float32),
        pltpu.VMEM_SHARED((n_pad, w), jnp.float32),
        pltpu.SemaphoreType.DMA,
        pltpu.SemaphoreType.DMA,
        pltpu.SemaphoreType.DMA,
        pltpu.SemaphoreType.DMA,
        pltpu.SemaphoreType.DMA,
        pltpu.SemaphoreType.DMA,
        pltpu.SemaphoreType.DMA,
        pltpu.SemaphoreType.DMA,
    ]

    def body(e2_h, h_h, acc_o, iga, igb, r0, r1, r2, zbuf, tbl, acc,
             sia, sib, sg0, sg1, sg2, ss0, ss1, ss2):
        c = lax.axis_index("c")
        s = lax.axis_index("s")

        pltpu.sync_copy(h_h.at[c, pl.ds(s * tbl_ps, tbl_ps)],
                        tbl.at[pl.ds(s * tbl_ps, tbl_ps)])

        z16 = jnp.zeros((16,), jnp.float32)
        for i in range(16):
            for j in range(w // 16):
                zbuf[i, pl.ds(j * 16, 16)] = z16

        base_row = s * rows_ps

        def zero_body(t, carry):
            pltpu.sync_copy(zbuf, acc.at[pl.ds(base_row + t * 16, 16)])
            return carry

        def pref(g, ig, sem):
            return pltpu.async_copy(e2_h.at[s, pl.ds(g * _G, _G)], ig, sem)

        def pref_wait(d):
            d.wait()

        lax.fori_loop(0, rows_ps // 16, zero_body, 0)
        pref_wait(pref(0, iga, sia))
        plsc.subcore_barrier()

        rb = (r0, r1, r2)
        sg = (sg0, sg1, sg2)
        ss = (ss0, ss1, ss2)

        def process_group(ig):
            dg = [None, None, None]
            ds = [None, None, None]
            dg[0] = pltpu.async_copy(tbl.at[ig.at[0, 0]], rb[0], sg[0])
            dg[1] = pltpu.async_copy(tbl.at[ig.at[1, 0]], rb[1], sg[1])
            for j in range(_G):
                b = j % 3
                dg[b].wait()
                ds[b] = pltpu.async_copy(rb[b], acc.at[ig.at[j, 1]], ss[b],
                                         add=True)
                if j + 2 < _G:
                    b2 = (j + 2) % 3
                    if ds[b2] is not None:
                        ds[b2].wait()
                    dg[b2] = pltpu.async_copy(tbl.at[ig.at[j + 2, 0]],
                                              rb[b2], sg[b2])
            for b in ((_G - 3) % 3, (_G - 2) % 3, (_G - 1) % 3):
                ds[b].wait()

        def outer_body(t, carry):
            ga = 2 * t
            dpb = pref(ga + 1, igb, sib)
            process_group(iga)
            pref_wait(dpb)
            dpa = pref(lax.rem(ga + 2, ngroups), iga, sia)
            process_group(igb)
            pref_wait(dpa)
            return carry

        lax.fori_loop(0, ngroups // 2, outer_body, 0)
        plsc.subcore_barrier()

        pltpu.sync_copy(acc.at[pl.ds(base_row, rows_ps)],
                        acc_o.at[c, pl.ds(base_row, rows_ps)])

    return pl.kernel(
        body,
        out_type=jax.ShapeDtypeStruct((_NC, n_pad, w), jnp.float32),
        mesh=mesh, scratch_types=scratch,
        compiler_params=pltpu.CompilerParams(use_tc_tiling_on_sc=False))


def _tc_layer1(P, x, Wl, bl, Wr, block_rows):
    n, d = x.shape
    wa = P.shape[2]
    db = d - wa
    hh = d // 2

    def body(p_ref, x_ref, wl_ref, bl_ref, wr_ref, o_ref):
        agg = jnp.concatenate([p_ref[0], p_ref[1, :, :db]], axis=1)
        cnt = jnp.mean(p_ref[1, :, db:], axis=1, keepdims=True)
        mean = agg / jnp.maximum(cnt, 1.0)
        h = jnp.dot(mean, wl_ref[...], preferred_element_type=jnp.float32)
        h = h + bl_ref[...]
        h = h + jnp.dot(x_ref[...], wr_ref[...], preferred_element_type=jnp.float32)
        h = jnp.maximum(h, 0.0)
        o_ref[0] = h[:, :hh]
        o_ref[1] = h[:, hh:]

    return pl.pallas_call(
        body,
        grid=(n // block_rows,),
        in_specs=[
            pl.BlockSpec((_NC, block_rows, wa), lambda i: (0, i, 0)),
            pl.BlockSpec((block_rows, d), lambda i: (i, 0)),
            pl.BlockSpec((d, d), lambda i: (0, 0)),
            pl.BlockSpec((1, d), lambda i: (0, 0)),
            pl.BlockSpec((d, d), lambda i: (0, 0)),
        ],
        out_specs=pl.BlockSpec((2, block_rows, hh), lambda i: (0, i, 0)),
        out_shape=jax.ShapeDtypeStruct((2, n, hh), jnp.float32),
    )(P, x, Wl, bl.reshape(1, d), Wr)


def _tc_layer2(P, C1, h2, x, Wl, bl, Wr, gamma, beta, block_rows):
    n, d = x.shape
    wc = C1.shape[2]
    db = d - wc
    hh = d // 2

    def body(p_ref, c_ref, h_ref, x_ref, wl_ref, bl_ref, wr_ref,
             g_ref, b_ref, o_ref):
        agg = jnp.concatenate([p_ref[0], p_ref[1]], axis=1)
        cnt = jnp.mean(c_ref[0, :, db:], axis=1, keepdims=True)
        mean = agg / jnp.maximum(cnt, 1.0)
        h = jnp.dot(mean, wl_ref[...], preferred_element_type=jnp.float32)
        h = h + bl_ref[...]
        h = h + jnp.dot(h_ref[0], wr_ref[0], preferred_element_type=jnp.float32)
        h = h + jnp.dot(h_ref[1], wr_ref[1], preferred_element_type=jnp.float32)
        h = h + x_ref[...]
        mu = jnp.mean(h, axis=1, keepdims=True)
        hc = h - mu
        var = jnp.mean(hc * hc, axis=1, keepdims=True)
        o_ref[...] = hc * lax.rsqrt(var + 1e-5) * g_ref[...] + b_ref[...]

    return pl.pallas_call(
        body,
        grid=(n // block_rows,),
        in_specs=[
            pl.BlockSpec((_NC, block_rows, hh), lambda i: (0, i, 0)),
            pl.BlockSpec((1, block_rows, wc), lambda i: (1, i, 0)),
            pl.BlockSpec((2, block_rows, hh), lambda i: (0, i, 0)),
            pl.BlockSpec((block_rows, d), lambda i: (i, 0)),
            pl.BlockSpec((d, d), lambda i: (0, 0)),
            pl.BlockSpec((1, d), lambda i: (0, 0)),
            pl.BlockSpec((2, hh, d), lambda i: (0, 0, 0)),
            pl.BlockSpec((1, d), lambda i: (0, 0)),
            pl.BlockSpec((1, d), lambda i: (0, 0)),
        ],
        out_specs=pl.BlockSpec((block_rows, d), lambda i: (i, 0)),
        out_shape=jax.ShapeDtypeStruct((n, d), jnp.float32),
    )(P, C1, h2, x, Wl, bl.reshape(1, d), Wr.reshape(2, hh, d),
      gamma.reshape(1, d), beta.reshape(1, d))


def kernel(x, edge_index, W1l, b1l, W1r, W2l, b2l, W2r, gamma, beta):
    n, d = x.shape
    e = edge_index.shape[1]

    e_pad = _round_up(e, _NS * _CHUNK * _G * 2)
    n_pad = _round_up(n + 1, 16 * _NS)

    src = edge_index[0]
    dst = edge_index[1]
    if e_pad != e:
        pad = e_pad - e
        src = jnp.concatenate([src, jnp.zeros((pad,), jnp.int32)])
        dst = jnp.concatenate([dst, jnp.full((pad,), n, jnp.int32)])
    nchunk = e_pad // (_NS * _CHUNK)
    e2 = jnp.stack([src.reshape(_NS, nchunk, _CHUNK),
                    dst.reshape(_NS, nchunk, _CHUNK)], axis=2)

    w1 = d + _CW
    ha = w1 // 2
    x2 = jnp.stack([x[:, :ha],
                    jnp.concatenate([x[:, ha:],
                                     jnp.ones((n, _CW), jnp.float32)], axis=1)])

    P1 = _make_sc_agg(n, n_pad, ha, e_pad)(e2, x2)
    h2 = _tc_layer1(P1, x, W1l, b1l, W1r, 1000)
    P2 = _make_sc_agg(n, n_pad, d // 2, e_pad)(e2, h2)
    return _tc_layer2(P2, P1, h2, x, W2l, b2l, W2r, gamma, beta, 1000)
